# baseline (device time: 107160 ns/iter reference)
import jax
import jax.numpy as jnp
from jax import lax
from jax.experimental import pallas as pl
from jax.experimental.pallas import tpu as pltpu

N_ROWS = 2048
N_COLS = 2048
N_DEV = 8
BLK = N_ROWS // N_DEV
HALF = N_COLS // 2

MESH = pltpu.DeviceIdType.MESH


def kernel(partial, resid, gamma):
    p2 = partial.reshape(N_ROWS, N_COLS)
    g2 = gamma.reshape(1, N_COLS)

    def body(p_ref, r_ref, g_ref, out_ref,
             p_mine, p_send_f32, r_mine, send_buf, recv_buf,
             copy_sems, send_sems, recv_sems):
        my_x = lax.axis_index("x")
        my_y = lax.axis_index("y")
        my_z = lax.axis_index("z")

        b_me = 4 * my_y + 2 * my_x + my_z
        b_pr = 4 * (1 - my_y) + 2 * my_x + my_z

        barrier = pltpu.get_barrier_semaphore()
        for nbr in ((my_x, 1 - my_y, my_z),
                    (1 - my_x, my_y, my_z),
                    (my_x, my_y, 1 - my_z)):
            pltpu.semaphore_signal(barrier, inc=1, device_id=nbr,
                                   device_id_type=MESH)
        pltpu.semaphore_wait(barrier, 3)

        cp_send = pltpu.make_async_copy(
            p_ref.at[pl.ds(b_pr * BLK, BLK), :], p_send_f32, copy_sems.at[0])
        cp_mine = pltpu.make_async_copy(
            p_ref.at[pl.ds(b_me * BLK, BLK), :], p_mine, copy_sems.at[1])
        cp_res = pltpu.make_async_copy(
            r_ref.at[pl.ds(b_me * BLK, BLK), :], r_mine, copy_sems.at[2])
        cp_send.start()
        cp_mine.start()
        cp_res.start()

        cp_send.wait()
        send_buf[...] = p_send_f32[...].astype(jnp.bfloat16)
        rdma_y = []
        for h in range(2):
            r = pltpu.make_async_remote_copy(
                src_ref=send_buf.at[:, pl.ds(h * HALF, HALF)],
                dst_ref=recv_buf.at[:, pl.ds(h * HALF, HALF)],
                send_sem=send_sems.at[0, h], recv_sem=recv_sems.at[0, h],
                device_id=(my_x, 1 - my_y, my_z), device_id_type=MESH)
            r.start()
            rdma_y.append(r)
        cp_mine.wait()
        cp_res.wait()
        for r in rdma_y:
            r.wait()

        y = p_mine[...] + recv_buf[...].astype(jnp.float32) + r_mine[...]
        ms = jnp.mean(y * y, axis=-1, keepdims=True)
        o = y * lax.rsqrt(ms + 1e-6) * g_ref[...]
        out_ref[pl.ds(b_me * BLK, BLK), :] = o.astype(jnp.bfloat16)

        starts = (b_me * BLK, (2 * my_y + my_x) * (2 * BLK), my_y * (4 * BLK))
        sizes = (BLK, 2 * BLK, 4 * BLK)
        partners = ((my_x, my_y, 1 - my_z),
                    (1 - my_x, my_y, my_z),
                    (my_x, 1 - my_y, my_z))
        for s in range(3):
            rdmas = []
            for h in range(2):
                r = pltpu.make_async_remote_copy(
                    src_ref=out_ref.at[pl.ds(starts[s], sizes[s]),
                                       pl.ds(h * HALF, HALF)],
                    dst_ref=out_ref.at[pl.ds(starts[s], sizes[s]),
                                       pl.ds(h * HALF, HALF)],
                    send_sem=send_sems.at[s + 1, h],
                    recv_sem=recv_sems.at[s + 1, h],
                    device_id=partners[s], device_id_type=MESH)
                r.start()
                rdmas.append(r)
            for r in rdmas:
                r.wait()

    return pl.pallas_call(
        body,
        out_shape=jax.ShapeDtypeStruct((N_ROWS, N_COLS), jnp.bfloat16),
        in_specs=[pl.BlockSpec(memory_space=pl.ANY),
                  pl.BlockSpec(memory_space=pl.ANY),
                  pl.BlockSpec(memory_space=pltpu.VMEM)],
        out_specs=pl.BlockSpec(memory_space=pltpu.VMEM),
        scratch_shapes=[
            pltpu.VMEM((BLK, N_COLS), jnp.float32),
            pltpu.VMEM((BLK, N_COLS), jnp.float32),
            pltpu.VMEM((BLK, N_COLS), jnp.float32),
            pltpu.VMEM((BLK, N_COLS), jnp.bfloat16),
            pltpu.VMEM((BLK, N_COLS), jnp.bfloat16),
            pltpu.SemaphoreType.DMA((3,)),
            pltpu.SemaphoreType.DMA((4, 2)),
            pltpu.SemaphoreType.DMA((4, 2)),
        ],
        compiler_params=pltpu.CompilerParams(collective_id=0),
    )(p2, resid, g2)


# device time: 58296 ns/iter; 1.8382x vs baseline; 1.8382x over previous
import jax
import jax.numpy as jnp
from jax import lax
from jax.experimental import pallas as pl
from jax.experimental.pallas import tpu as pltpu

N_ROWS = 2048
N_COLS = 2048
N_DEV = 8
BLK = N_ROWS // N_DEV

COLS = ((0, 768), (768, 640), (1408, 640))
ORDERS = (("z", "x", "y"), ("x", "y", "z"), ("y", "z", "x"))
BITV = {"z": 1, "x": 2, "y": 4}

MESH = pltpu.DeviceIdType.MESH


def kernel(partial, resid, gamma):
    p2 = partial.reshape(N_ROWS, N_COLS)
    g2 = gamma.reshape(1, N_COLS)

    def body(p_ref, r_ref, g_ref, out_ref,
             p_mine, p_send_f32, r_mine, send_buf, recv_buf,
             copy_sems, send_sems, recv_sems):
        my_x = lax.axis_index("x")
        my_y = lax.axis_index("y")
        my_z = lax.axis_index("z")

        b_me = 4 * my_y + 2 * my_x + my_z
        b_pr = 4 * (1 - my_y) + 2 * my_x + my_z

        def partner_of(axis):
            if axis == "x":
                return (1 - my_x, my_y, my_z)
            if axis == "y":
                return (my_x, 1 - my_y, my_z)
            return (my_x, my_y, 1 - my_z)

        barrier = pltpu.get_barrier_semaphore()
        for axis in ("x", "y", "z"):
            pltpu.semaphore_signal(barrier, inc=1, device_id=partner_of(axis),
                                   device_id_type=MESH)
        pltpu.semaphore_wait(barrier, 3)

        cp_send = pltpu.make_async_copy(
            p_ref.at[pl.ds(b_pr * BLK, BLK), :], p_send_f32, copy_sems.at[0])
        cp_mine = pltpu.make_async_copy(
            p_ref.at[pl.ds(b_me * BLK, BLK), :], p_mine, copy_sems.at[1])
        cp_res = pltpu.make_async_copy(
            r_ref.at[pl.ds(b_me * BLK, BLK), :], r_mine, copy_sems.at[2])
        cp_send.start()
        cp_mine.start()
        cp_res.start()

        cp_send.wait()
        send_buf[...] = p_send_f32[...].astype(jnp.bfloat16)
        rdma_y = pltpu.make_async_remote_copy(
            src_ref=send_buf, dst_ref=recv_buf,
            send_sem=send_sems.at[21], recv_sem=recv_sems.at[21],
            device_id=partner_of("y"), device_id_type=MESH)
        rdma_y.start()

        cp_mine.wait()
        cp_res.wait()
        p_mine[...] = p_mine[...] + r_mine[...]
        rdma_y.wait()

        y = p_mine[...] + recv_buf[...].astype(jnp.float32)
        ms = jnp.mean(y * y, axis=-1, keepdims=True)
        o = y * lax.rsqrt(ms + 1e-6) * g_ref[...]
        out_ref[pl.ds(b_me * BLK, BLK), :] = o.astype(jnp.bfloat16)

        sem_i = 0
        held = [[b_me], [b_me], [b_me]]
        for s in range(3):
            rdmas = []
            for p in range(3):
                axis = ORDERS[p][s]
                tgt = partner_of(axis)
                c0, cw = COLS[p]
                for b in held[p]:
                    r = pltpu.make_async_remote_copy(
                        src_ref=out_ref.at[pl.ds(b * BLK, BLK), pl.ds(c0, cw)],
                        dst_ref=out_ref.at[pl.ds(b * BLK, BLK), pl.ds(c0, cw)],
                        send_sem=send_sems.at[sem_i],
                        recv_sem=recv_sems.at[sem_i],
                        device_id=tgt, device_id_type=MESH)
                    r.start()
                    rdmas.append(r)
                    sem_i += 1
                held[p] = held[p] + [b ^ BITV[axis] for b in held[p]]
            for r in rdmas:
                r.wait()

    return pl.pallas_call(
        body,
        out_shape=jax.ShapeDtypeStruct((N_ROWS, N_COLS), jnp.bfloat16),
        in_specs=[pl.BlockSpec(memory_space=pl.ANY),
                  pl.BlockSpec(memory_space=pl.ANY),
                  pl.BlockSpec(memory_space=pltpu.VMEM)],
        out_specs=pl.BlockSpec(memory_space=pltpu.VMEM),
        scratch_shapes=[
            pltpu.VMEM((BLK, N_COLS), jnp.float32),
            pltpu.VMEM((BLK, N_COLS), jnp.float32),
            pltpu.VMEM((BLK, N_COLS), jnp.float32),
            pltpu.VMEM((BLK, N_COLS), jnp.bfloat16),
            pltpu.VMEM((BLK, N_COLS), jnp.bfloat16),
            pltpu.SemaphoreType.DMA((3,)),
            pltpu.SemaphoreType.DMA((22,)),
            pltpu.SemaphoreType.DMA((22,)),
        ],
        compiler_params=pltpu.CompilerParams(collective_id=0),
    )(p2, resid, g2)


# device time: 56039 ns/iter; 1.9122x vs baseline; 1.0403x over previous
import jax
import jax.numpy as jnp
from jax import lax
from jax.experimental import pallas as pl
from jax.experimental.pallas import tpu as pltpu

N_ROWS = 2048
N_COLS = 2048
N_DEV = 8
BLK = N_ROWS // N_DEV

COLS = ((0, 768), (768, 640), (1408, 640))
ORDERS = (("z", "x", "y"), ("x", "y", "z"), ("y", "z", "x"))
BITV = {"z": 1, "x": 2, "y": 4}

MESH = pltpu.DeviceIdType.MESH


def kernel(partial, resid, gamma):
    p2 = partial.reshape(N_ROWS, N_COLS)
    g2 = gamma.reshape(1, N_COLS)

    def body(p_ref, r_ref, g_ref, out_ref,
             p_mine, p_send_f32, r_mine, send_buf, recv_buf,
             copy_sems, send_sems, recv_sems):
        my_x = lax.axis_index("x")
        my_y = lax.axis_index("y")
        my_z = lax.axis_index("z")

        b_me = 4 * my_y + 2 * my_x + my_z
        b_pr = 4 * (1 - my_y) + 2 * my_x + my_z

        def partner_of(axis):
            if axis == "x":
                return (1 - my_x, my_y, my_z)
            if axis == "y":
                return (my_x, 1 - my_y, my_z)
            return (my_x, my_y, 1 - my_z)

        barrier = pltpu.get_barrier_semaphore()
        for axis in ("x", "y", "z"):
            pltpu.semaphore_signal(barrier, inc=1, device_id=partner_of(axis),
                                   device_id_type=MESH)
        pltpu.semaphore_wait(barrier, 3)

        cp_send = pltpu.make_async_copy(
            p_ref.at[pl.ds(b_pr * BLK, BLK), :], p_send_f32, copy_sems.at[0])
        cp_mine = pltpu.make_async_copy(
            p_ref.at[pl.ds(b_me * BLK, BLK), :], p_mine, copy_sems.at[1])
        cp_res = pltpu.make_async_copy(
            r_ref.at[pl.ds(b_me * BLK, BLK), :], r_mine, copy_sems.at[2])
        cp_send.start()
        cp_mine.start()
        cp_res.start()

        cp_send.wait()
        send_buf[...] = p_send_f32[...].astype(jnp.bfloat16)
        ph1 = []
        for c in range(3):
            c0, cw = COLS[c]
            r = pltpu.make_async_remote_copy(
                src_ref=send_buf.at[:, pl.ds(c0, cw)],
                dst_ref=recv_buf.at[:, pl.ds(c0, cw)],
                send_sem=send_sems.at[21 + c], recv_sem=recv_sems.at[21 + c],
                device_id=partner_of("y"), device_id_type=MESH)
            r.start()
            ph1.append(r)

        cp_mine.wait()
        cp_res.wait()
        p_mine[...] = p_mine[...] + r_mine[...]

        sem_i = 0
        held = [[b_me], [b_me], [b_me]]
        inflight = [[], [], []]
        for p in range(3):
            ph1[p].wait()
            c0, cw = COLS[p]
            yc = (p_mine[:, c0:c0 + cw]
                  + recv_buf[:, c0:c0 + cw].astype(jnp.float32))
            out_ref[pl.ds(b_me * BLK, BLK), c0:c0 + cw] = yc.astype(
                jnp.bfloat16)
            axis = ORDERS[p][0]
            r = pltpu.make_async_remote_copy(
                src_ref=out_ref.at[pl.ds(b_me * BLK, BLK), pl.ds(c0, cw)],
                dst_ref=out_ref.at[pl.ds(b_me * BLK, BLK), pl.ds(c0, cw)],
                send_sem=send_sems.at[sem_i], recv_sem=recv_sems.at[sem_i],
                device_id=partner_of(axis), device_id_type=MESH)
            r.start()
            sem_i += 1
            inflight[p] = [r]
            held[p] = held[p] + [b ^ BITV[axis] for b in held[p]]

        for s in (1, 2):
            for p in range(3):
                for r in inflight[p]:
                    r.wait()
                axis = ORDERS[p][s]
                tgt = partner_of(axis)
                c0, cw = COLS[p]
                new = []
                for b in held[p]:
                    r = pltpu.make_async_remote_copy(
                        src_ref=out_ref.at[pl.ds(b * BLK, BLK),
                                           pl.ds(c0, cw)],
                        dst_ref=out_ref.at[pl.ds(b * BLK, BLK),
                                           pl.ds(c0, cw)],
                        send_sem=send_sems.at[sem_i],
                        recv_sem=recv_sems.at[sem_i],
                        device_id=tgt, device_id_type=MESH)
                    r.start()
                    new.append(r)
                    sem_i += 1
                held[p] = held[p] + [b ^ BITV[axis] for b in held[p]]
                inflight[p] = new
        for p in range(3):
            for r in inflight[p]:
                r.wait()

        for i in range(N_DEV):
            rows = slice(i * BLK, (i + 1) * BLK)
            o = out_ref[rows, :].astype(jnp.float32)
            ms = jnp.mean(o * o, axis=-1, keepdims=True)
            out_ref[rows, :] = (o * lax.rsqrt(ms + 1e-6)
                                * g_ref[...]).astype(jnp.bfloat16)

    return pl.pallas_call(
        body,
        out_shape=jax.ShapeDtypeStruct((N_ROWS, N_COLS), jnp.bfloat16),
        in_specs=[pl.BlockSpec(memory_space=pl.ANY),
                  pl.BlockSpec(memory_space=pl.ANY),
                  pl.BlockSpec(memory_space=pltpu.VMEM)],
        out_specs=pl.BlockSpec(memory_space=pltpu.VMEM),
        scratch_shapes=[
            pltpu.VMEM((BLK, N_COLS), jnp.float32),
            pltpu.VMEM((BLK, N_COLS), jnp.float32),
            pltpu.VMEM((BLK, N_COLS), jnp.float32),
            pltpu.VMEM((BLK, N_COLS), jnp.bfloat16),
            pltpu.VMEM((BLK, N_COLS), jnp.bfloat16),
            pltpu.SemaphoreType.DMA((3,)),
            pltpu.SemaphoreType.DMA((24,)),
            pltpu.SemaphoreType.DMA((24,)),
        ],
        compiler_params=pltpu.CompilerParams(collective_id=0),
    )(p2, resid, g2)
